# Initial kernel scaffold; baseline (speedup 1.0000x reference)
import jax
import jax.numpy as jnp
from jax.experimental import pallas as pl


def kernel(conv0_w, conv0_b, conv1_w, conv1_b, conv2_w, conv2_b, conv3_w, conv3_b, conv4_w, conv4_b, conv5_w, conv5_b, conv6_w, conv6_b, conv7_w, conv7_b, conv8_w, conv8_b, conv9_w, conv9_b, conv10_w, conv10_b, conv11_w, conv11_b, conv12_w, conv12_b, fc1_w, fc1_b, fc2_w, fc2_b, fc3_w, fc3_b, x_nhwc, drop_key):
    raise NotImplementedError("write your pallas kernel here")


# speedup vs baseline: 1.3668x; 1.3668x over previous
import functools

import jax
import jax.numpy as jnp
from jax.experimental import pallas as pl
from jax.experimental.pallas import tpu as pltpu


def _conv_body(x_ref, w_ref, b_ref, o_ref, *, H, W, tr, tco, group, pool,
               pad_out):
    Wp = W + 2
    m2 = tr * Wp
    nt = H // tr
    H2 = H // 2 if pool else H
    W2 = W // 2 if pool else W
    Wp2 = W2 + 2
    if pad_out:
        zrow = jnp.zeros((2 * Wp2, tco), o_ref.dtype)
        o_ref[0, pl.ds(0, 2 * Wp2), :] = zrow
        o_ref[0, pl.ds((H2 + 2) * Wp2, 2 * Wp2), :] = zrow
    for i in range(nt):
        base = i * tr * Wp
        if group == 9:
            lhs = jnp.concatenate(
                [x_ref[0, pl.ds(base + (dy + 1) * Wp + dx - 1, m2), :]
                 for dy in range(3) for dx in range(3)], axis=-1)
            acc = jnp.dot(lhs, w_ref[0], preferred_element_type=jnp.float32)
        elif group == 3:
            acc = jnp.zeros((m2, tco), jnp.float32)
            for dy in range(3):
                lhs = jnp.concatenate(
                    [x_ref[0, pl.ds(base + (dy + 1) * Wp + dx - 1, m2), :]
                     for dx in range(3)], axis=-1)
                acc = acc + jnp.dot(lhs, w_ref[dy],
                                    preferred_element_type=jnp.float32)
        else:
            acc = jnp.zeros((m2, tco), jnp.float32)
            for dy in range(3):
                for dx in range(3):
                    acc = acc + jnp.dot(
                        x_ref[0, pl.ds(base + (dy + 1) * Wp + dx - 1, m2), :],
                        w_ref[dy * 3 + dx],
                        preferred_element_type=jnp.float32)
        y = jnp.maximum(acc + b_ref[...], 0.0)
        if pool:
            tr2 = tr // 2
            v = y.reshape(tr, Wp, tco)[:, 1:W + 1, :]
            v = v.reshape(tr2, 2, W, tco)
            v = jnp.maximum(v[:, 0], v[:, 1])
            v = v.reshape(tr2, W2, 2, tco)
            p = jnp.maximum(v[:, :, 0], v[:, :, 1])
            if pad_out:
                pv = jnp.pad(p, ((0, 0), (1, 1), (0, 0)))
                o_ref[0, pl.ds((i * tr2 + 2) * Wp2, tr2 * Wp2), :] = (
                    pv.reshape(tr2 * Wp2, tco).astype(o_ref.dtype))
            else:
                o_ref[0, pl.ds(i * tr2 * W2, tr2 * W2), :] = (
                    p.reshape(tr2 * W2, tco).astype(o_ref.dtype))
        else:
            col = jax.lax.broadcasted_iota(jnp.int32, (m2, 1), 0) % Wp
            mask = jnp.logical_and(col != 0, col != Wp - 1)
            yv = jnp.where(mask, y, 0.0)
            o_ref[0, pl.ds((i * tr + 2) * Wp, m2), :] = yv.astype(o_ref.dtype)


def _conv_band_body(x_ref, w_ref, b_ref, o_ref, *, W, tr, tco, group, pool):
    Wp = W + 2
    m2 = tr * Wp
    if group == 9:
        lhs = jnp.concatenate(
            [x_ref[0, 0, pl.ds((dy + 1) * Wp + dx - 1, m2), :]
             for dy in range(3) for dx in range(3)], axis=-1)
        acc = jnp.dot(lhs, w_ref[0], preferred_element_type=jnp.float32)
    else:
        acc = jnp.zeros((m2, tco), jnp.float32)
        for dy in range(3):
            lhs = jnp.concatenate(
                [x_ref[0, 0, pl.ds((dy + 1) * Wp + dx - 1, m2), :]
                 for dx in range(3)], axis=-1)
            acc = acc + jnp.dot(lhs, w_ref[dy],
                                preferred_element_type=jnp.float32)
    y = jnp.maximum(acc + b_ref[...], 0.0)
    if pool:
        tr2 = tr // 2
        W2 = W // 2
        v = y.reshape(tr, Wp, tco)[:, 1:W + 1, :]
        v = v.reshape(tr2, 2, W, tco)
        v = jnp.maximum(v[:, 0], v[:, 1])
        v = v.reshape(tr2, W2, 2, tco)
        p = jnp.maximum(v[:, :, 0], v[:, :, 1])
        pv = jnp.pad(p, ((0, 0), (1, 1), (0, 0)))
        o_ref[0, 0] = pv.reshape(tr2 * (W2 + 2), tco).astype(o_ref.dtype)
    else:
        col = jax.lax.broadcasted_iota(jnp.int32, (m2, 1), 0) % Wp
        mask = jnp.logical_and(col != 0, col != Wp - 1)
        o_ref[0, 0] = jnp.where(mask, y, 0.0).astype(o_ref.dtype)


def _conv_banded(x, w, b, *, H, W, tr, pool, group):
    N, nt = x.shape[0], x.shape[1]
    cin = x.shape[-1]
    cout = w.shape[-1]
    tco = min(cout, 256)
    Wp = W + 2
    if pool:
        orows = (tr // 2) * (W // 2 + 2)
    else:
        orows = tr * Wp
    if group == 9:
        wm = w.reshape(1, 9 * cin, cout).astype(jnp.bfloat16)
        wspec = pl.BlockSpec((1, 9 * cin, tco), lambda n, c: (0, 0, 0))
    else:
        wm = w.reshape(3, 3 * cin, cout).astype(jnp.bfloat16)
        wspec = pl.BlockSpec((3, 3 * cin, tco), lambda n, c: (0, 0, 0))
    out = pl.pallas_call(
        functools.partial(_conv_band_body, W=W, tr=tr, tco=tco, group=group,
                          pool=pool),
        out_shape=jax.ShapeDtypeStruct((N, nt, orows, cout), jnp.bfloat16),
        grid=(N, nt),
        in_specs=[
            pl.BlockSpec((1, 1, (tr + 4) * Wp, cin),
                         lambda n, c: (n, c, 0, 0)),
            wspec,
            pl.BlockSpec((1, tco), lambda n, c: (0, 0)),
        ],
        out_specs=pl.BlockSpec((1, 1, orows, tco), lambda n, c: (n, c, 0, 0)),
        compiler_params=pltpu.CompilerParams(
            dimension_semantics=("parallel", "parallel")),
    )(x, wm, b.reshape(1, cout).astype(jnp.float32))
    return out.reshape(N, nt * orows, cout)


def _bands(xp, tr):
    N, Hp, Wp, C = xp.shape
    nt = (Hp - 4) // tr
    t = jnp.stack([xp[:, c * tr:c * tr + tr + 4] for c in range(nt)], axis=1)
    return t.reshape(N, nt, (tr + 4) * Wp, C)


def _conv(x, w, b, *, H, W, pool, pad_out, group):
    N = x.shape[0]
    cin = x.shape[-1]
    cout = w.shape[-1]
    tco = min(cout, 256)
    if H % 28 == 0:
        tr = 28
    elif H % 14 == 0:
        tr = 14
    else:
        tr = H
    Wp = W + 2
    Hp = H + 4
    H2 = H // 2 if pool else H
    W2 = W // 2 if pool else W
    out_rows = (H2 + 4) * (W2 + 2) if pad_out else H2 * W2
    if group == 9:
        wm = w.reshape(1, 9 * cin, cout).astype(jnp.bfloat16)
        wspec = pl.BlockSpec((1, 9 * cin, tco), lambda n, j: (0, 0, j))
    elif group == 3:
        wm = w.reshape(3, 3 * cin, cout).astype(jnp.bfloat16)
        wspec = pl.BlockSpec((3, 3 * cin, tco), lambda n, j: (0, 0, j))
    else:
        wm = w.reshape(9, cin, cout).astype(jnp.bfloat16)
        wspec = pl.BlockSpec((9, cin, tco), lambda n, j: (0, 0, j))
    return pl.pallas_call(
        functools.partial(_conv_body, H=H, W=W, tr=tr, tco=tco, group=group,
                          pool=pool, pad_out=pad_out),
        out_shape=jax.ShapeDtypeStruct((N, out_rows, cout), jnp.bfloat16),
        grid=(N, cout // tco),
        in_specs=[
            pl.BlockSpec((1, Hp * Wp, cin), lambda n, j: (n, 0, 0)),
            wspec,
            pl.BlockSpec((1, tco), lambda n, j: (0, j)),
        ],
        out_specs=pl.BlockSpec((1, out_rows, tco), lambda n, j: (n, 0, j)),
        compiler_params=pltpu.CompilerParams(
            dimension_semantics=("parallel", "arbitrary")),
    )(x, wm, b.reshape(1, cout).astype(jnp.float32))


def _fc_body(x_ref, w_ref, b_ref, s_ref, o_ref, *, relu):
    y = jnp.dot(x_ref[...], w_ref[...],
                preferred_element_type=jnp.float32) + b_ref[...]
    if relu:
        y = jnp.maximum(y, 0.0)
    o_ref[...] = (y * s_ref[...]).astype(o_ref.dtype)


def _fc(x, w, b, scale, *, relu, tn, out_dtype):
    B, K = x.shape
    Nout = w.shape[1]
    return pl.pallas_call(
        functools.partial(_fc_body, relu=relu),
        out_shape=jax.ShapeDtypeStruct((B, Nout), out_dtype),
        grid=(Nout // tn,),
        in_specs=[
            pl.BlockSpec((B, K), lambda j: (0, 0)),
            pl.BlockSpec((K, tn), lambda j: (0, j)),
            pl.BlockSpec((1, tn), lambda j: (0, j)),
            pl.BlockSpec((B, tn), lambda j: (0, j)),
        ],
        out_specs=pl.BlockSpec((B, tn), lambda j: (0, j)),
        compiler_params=pltpu.CompilerParams(
            dimension_semantics=("parallel",)),
    )(x, w, b.reshape(1, Nout).astype(jnp.float32), scale.astype(jnp.float32))


def kernel(conv0_w, conv0_b, conv1_w, conv1_b, conv2_w, conv2_b, conv3_w,
           conv3_b, conv4_w, conv4_b, conv5_w, conv5_b, conv6_w, conv6_b,
           conv7_w, conv7_b, conv8_w, conv8_b, conv9_w, conv9_b, conv10_w,
           conv10_b, conv11_w, conv11_b, conv12_w, conv12_b, fc1_w, fc1_b,
           fc2_w, fc2_b, fc3_w, fc3_b, x_nhwc, drop_key):
    convs = [(conv0_w, conv0_b), (conv1_w, conv1_b), (conv2_w, conv2_b),
             (conv3_w, conv3_b), (conv4_w, conv4_b), (conv5_w, conv5_b),
             (conv6_w, conv6_b), (conv7_w, conv7_b), (conv8_w, conv8_b),
             (conv9_w, conv9_b), (conv10_w, conv10_b), (conv11_w, conv11_b),
             (conv12_w, conv12_b)]
    N, H, W, _ = x_nhwc.shape

    xp = jnp.pad(x_nhwc.astype(jnp.bfloat16),
                 ((0, 0), (2, 2), (1, 1), (0, 0)))
    x = _conv_banded(_bands(xp, 28), conv0_w, conv0_b, H=224, W=224, tr=28,
                     pool=False, group=9)
    x = jnp.pad(x.reshape(N, 224, 226, 64), ((0, 0), (2, 2), (0, 0), (0, 0)))
    x = _conv_banded(_bands(x, 28), conv1_w, conv1_b, H=224, W=224, tr=28,
                     pool=True, group=3)
    x = jnp.pad(x.reshape(N, 112, 114, 64), ((0, 0), (2, 2), (0, 0), (0, 0)))
    x = x.reshape(N, 116 * 114, 64)

    cfg = [
        (112, 112, False, 3),
        (112, 112, True, 3),
        (56, 56, False, 3),
        (56, 56, False, 1),
        (56, 56, True, 1),
        (28, 28, False, 1),
        (28, 28, False, 1),
        (28, 28, True, 1),
        (14, 14, False, 1),
        (14, 14, False, 1),
        (14, 14, True, 1),
    ]
    for li, (h, w_sp, pool, group) in enumerate(cfg):
        wq, bq = convs[li + 2]
        pad_out = li != 10
        x = _conv(x, wq, bq, H=h, W=w_sp, pool=pool, pad_out=pad_out,
                  group=group)

    x = jnp.transpose(x, (0, 2, 1)).reshape(N, 512 * 7 * 7)

    k1, k2 = jax.random.split(drop_key)
    s1 = jax.random.bernoulli(k1, 0.5, (N, 4096)).astype(jnp.float32) / 0.5
    s2 = jax.random.bernoulli(k2, 0.5, (N, 4096)).astype(jnp.float32) / 0.5

    x = _fc(x, fc1_w, fc1_b, s1, relu=True, tn=256, out_dtype=jnp.bfloat16)
    x = _fc(x, fc2_w, fc2_b, s2, relu=True, tn=256, out_dtype=jnp.bfloat16)
    ones = jnp.ones((N, fc3_w.shape[1]), jnp.float32)
    x = _fc(x, fc3_w, fc3_b, ones, relu=False, tn=fc3_w.shape[1],
            out_dtype=jnp.float32)
    return x


# speedup vs baseline: 1.4970x; 1.0953x over previous
import functools

import jax
import jax.numpy as jnp
from jax.experimental import pallas as pl
from jax.experimental.pallas import tpu as pltpu


def _conv_body(x_ref, w_ref, b_ref, o_ref, *, H, W, tr, tco, group, pool,
               pad_out):
    Wp = W + 2
    m2 = tr * Wp
    nt = H // tr
    H2 = H // 2 if pool else H
    W2 = W // 2 if pool else W
    Wp2 = W2 + 2
    if pad_out:
        zrow = jnp.zeros((2 * Wp2, tco), o_ref.dtype)
        o_ref[0, pl.ds(0, 2 * Wp2), :] = zrow
        o_ref[0, pl.ds((H2 + 2) * Wp2, 2 * Wp2), :] = zrow
    for i in range(nt):
        base = i * tr * Wp
        if group == 9:
            lhs = jnp.concatenate(
                [x_ref[0, pl.ds(base + (dy + 1) * Wp + dx - 1, m2), :]
                 for dy in range(3) for dx in range(3)], axis=-1)
            acc = jnp.dot(lhs, w_ref[0], preferred_element_type=jnp.float32)
        elif group == 3:
            acc = jnp.zeros((m2, tco), jnp.float32)
            for dy in range(3):
                lhs = jnp.concatenate(
                    [x_ref[0, pl.ds(base + (dy + 1) * Wp + dx - 1, m2), :]
                     for dx in range(3)], axis=-1)
                acc = acc + jnp.dot(lhs, w_ref[dy],
                                    preferred_element_type=jnp.float32)
        else:
            acc = jnp.zeros((m2, tco), jnp.float32)
            for dy in range(3):
                for dx in range(3):
                    acc = acc + jnp.dot(
                        x_ref[0, pl.ds(base + (dy + 1) * Wp + dx - 1, m2), :],
                        w_ref[dy * 3 + dx],
                        preferred_element_type=jnp.float32)
        y = jnp.maximum(acc + b_ref[...], 0.0)
        if pool:
            tr2 = tr // 2
            v = y.reshape(tr, Wp, tco)[:, 1:W + 1, :]
            v = v.reshape(tr2, 2, W, tco)
            v = jnp.maximum(v[:, 0], v[:, 1])
            v = v.reshape(tr2, W2, 2, tco)
            p = jnp.maximum(v[:, :, 0], v[:, :, 1])
            if pad_out:
                pv = jnp.pad(p, ((0, 0), (1, 1), (0, 0)))
                o_ref[0, pl.ds((i * tr2 + 2) * Wp2, tr2 * Wp2), :] = (
                    pv.reshape(tr2 * Wp2, tco).astype(o_ref.dtype))
            else:
                o_ref[0, pl.ds(i * tr2 * W2, tr2 * W2), :] = (
                    p.reshape(tr2 * W2, tco).astype(o_ref.dtype))
        else:
            col = jax.lax.broadcasted_iota(jnp.int32, (m2, 1), 0) % Wp
            mask = jnp.logical_and(col != 0, col != Wp - 1)
            yv = jnp.where(mask, y, 0.0)
            o_ref[0, pl.ds((i * tr + 2) * Wp, m2), :] = yv.astype(o_ref.dtype)


def _conv0_body(x_ref, w_ref, b_ref, o_ref, *, Wp):
    y = jnp.maximum(
        jnp.dot(x_ref[0], w_ref[...], preferred_element_type=jnp.float32)
        + b_ref[...], 0.0)
    m = y.shape[0]
    col = jax.lax.broadcasted_iota(jnp.int32, (m, 1), 0) % Wp
    mask = jnp.logical_and(col != 0, col != Wp - 1)
    o_ref[0] = jnp.where(mask, y, 0.0).astype(o_ref.dtype)


def _conv0(x_nhwc, w, b):
    N = x_nhwc.shape[0]
    Wp = 226
    xq = jnp.pad(x_nhwc.astype(jnp.bfloat16), ((0, 0), (1, 1), (2, 2), (0, 0)))
    pat = jnp.concatenate(
        [xq[:, dy:dy + 224, dx:dx + 226, :] for dy in range(3)
         for dx in range(3)], axis=-1)
    pat = pat.reshape(N, 224 * 226, 27)
    mrows = 28 * 226
    nr = (224 * 226) // mrows
    return pl.pallas_call(
        functools.partial(_conv0_body, Wp=Wp),
        out_shape=jax.ShapeDtypeStruct((N, 224 * 226, 64), jnp.bfloat16),
        grid=(N, nr),
        in_specs=[
            pl.BlockSpec((1, mrows, 27), lambda n, r: (n, r, 0)),
            pl.BlockSpec((27, 64), lambda n, r: (0, 0)),
            pl.BlockSpec((1, 64), lambda n, r: (0, 0)),
        ],
        out_specs=pl.BlockSpec((1, mrows, 64), lambda n, r: (n, r, 0)),
        compiler_params=pltpu.CompilerParams(
            dimension_semantics=("parallel", "parallel")),
    )(pat, w.reshape(27, 64).astype(jnp.bfloat16),
      b.reshape(1, 64).astype(jnp.float32))


def _conv_band_body(x_ref, w_ref, b_ref, o_ref, *, W, tr, tco, group, pool):
    Wp = W + 2
    m2 = tr * Wp
    if group == 9:
        lhs = jnp.concatenate(
            [x_ref[0, 0, pl.ds((dy + 1) * Wp + dx - 1, m2), :]
             for dy in range(3) for dx in range(3)], axis=-1)
        acc = jnp.dot(lhs, w_ref[0], preferred_element_type=jnp.float32)
    else:
        acc = jnp.zeros((m2, tco), jnp.float32)
        for dy in range(3):
            lhs = jnp.concatenate(
                [x_ref[0, 0, pl.ds((dy + 1) * Wp + dx - 1, m2), :]
                 for dx in range(3)], axis=-1)
            acc = acc + jnp.dot(lhs, w_ref[dy],
                                preferred_element_type=jnp.float32)
    y = jnp.maximum(acc + b_ref[...], 0.0)
    if pool:
        tr2 = tr // 2
        W2 = W // 2
        v = y.reshape(tr, Wp, tco)[:, 1:W + 1, :]
        v = v.reshape(tr2, 2, W, tco)
        v = jnp.maximum(v[:, 0], v[:, 1])
        v = v.reshape(tr2, W2, 2, tco)
        p = jnp.maximum(v[:, :, 0], v[:, :, 1])
        pv = jnp.pad(p, ((0, 0), (1, 1), (0, 0)))
        o_ref[0, 0] = pv.reshape(tr2 * (W2 + 2), tco).astype(o_ref.dtype)
    else:
        col = jax.lax.broadcasted_iota(jnp.int32, (m2, 1), 0) % Wp
        mask = jnp.logical_and(col != 0, col != Wp - 1)
        o_ref[0, 0] = jnp.where(mask, y, 0.0).astype(o_ref.dtype)


def _conv_banded(x, w, b, *, H, W, tr, pool, group):
    N, nt = x.shape[0], x.shape[1]
    cin = x.shape[-1]
    cout = w.shape[-1]
    tco = min(cout, 256)
    Wp = W + 2
    if pool:
        orows = (tr // 2) * (W // 2 + 2)
    else:
        orows = tr * Wp
    if group == 9:
        wm = w.reshape(1, 9 * cin, cout).astype(jnp.bfloat16)
        wspec = pl.BlockSpec((1, 9 * cin, tco), lambda n, c: (0, 0, 0))
    else:
        wm = w.reshape(3, 3 * cin, cout).astype(jnp.bfloat16)
        wspec = pl.BlockSpec((3, 3 * cin, tco), lambda n, c: (0, 0, 0))
    out = pl.pallas_call(
        functools.partial(_conv_band_body, W=W, tr=tr, tco=tco, group=group,
                          pool=pool),
        out_shape=jax.ShapeDtypeStruct((N, nt, orows, cout), jnp.bfloat16),
        grid=(N, nt),
        in_specs=[
            pl.BlockSpec((1, 1, (tr + 4) * Wp, cin),
                         lambda n, c: (n, c, 0, 0)),
            wspec,
            pl.BlockSpec((1, tco), lambda n, c: (0, 0)),
        ],
        out_specs=pl.BlockSpec((1, 1, orows, tco), lambda n, c: (n, c, 0, 0)),
        compiler_params=pltpu.CompilerParams(
            dimension_semantics=("parallel", "parallel")),
    )(x, wm, b.reshape(1, cout).astype(jnp.float32))
    return out.reshape(N, nt * orows, cout)


def _bands(xp, tr):
    N, Hp, Wp, C = xp.shape
    nt = (Hp - 4) // tr
    t = jnp.stack([xp[:, c * tr:c * tr + tr + 4] for c in range(nt)], axis=1)
    return t.reshape(N, nt, (tr + 4) * Wp, C)


def _conv(x, w, b, *, H, W, pool, pad_out, group):
    N = x.shape[0]
    cin = x.shape[-1]
    cout = w.shape[-1]
    tco = min(cout, 256)
    if H % 28 == 0:
        tr = 28
    elif H % 14 == 0:
        tr = 14
    else:
        tr = H
    Wp = W + 2
    Hp = H + 4
    H2 = H // 2 if pool else H
    W2 = W // 2 if pool else W
    out_rows = (H2 + 4) * (W2 + 2) if pad_out else H2 * W2
    if group == 9:
        wm = w.reshape(1, 9 * cin, cout).astype(jnp.bfloat16)
        wspec = pl.BlockSpec((1, 9 * cin, tco), lambda n, j: (0, 0, j))
    elif group == 3:
        wm = w.reshape(3, 3 * cin, cout).astype(jnp.bfloat16)
        wspec = pl.BlockSpec((3, 3 * cin, tco), lambda n, j: (0, 0, j))
    else:
        wm = w.reshape(9, cin, cout).astype(jnp.bfloat16)
        wspec = pl.BlockSpec((9, cin, tco), lambda n, j: (0, 0, j))
    return pl.pallas_call(
        functools.partial(_conv_body, H=H, W=W, tr=tr, tco=tco, group=group,
                          pool=pool, pad_out=pad_out),
        out_shape=jax.ShapeDtypeStruct((N, out_rows, cout), jnp.bfloat16),
        grid=(N, cout // tco),
        in_specs=[
            pl.BlockSpec((1, Hp * Wp, cin), lambda n, j: (n, 0, 0)),
            wspec,
            pl.BlockSpec((1, tco), lambda n, j: (0, j)),
        ],
        out_specs=pl.BlockSpec((1, out_rows, tco), lambda n, j: (n, 0, j)),
        compiler_params=pltpu.CompilerParams(
            dimension_semantics=("parallel", "arbitrary")),
    )(x, wm, b.reshape(1, cout).astype(jnp.float32))


def _fc_body(x_ref, w_ref, b_ref, s_ref, o_ref, *, relu):
    y = jnp.dot(x_ref[...], w_ref[...],
                preferred_element_type=jnp.float32) + b_ref[...]
    if relu:
        y = jnp.maximum(y, 0.0)
    o_ref[...] = (y * s_ref[...]).astype(o_ref.dtype)


def _fc(x, w, b, scale, *, relu, tn, out_dtype):
    B, K = x.shape
    Nout = w.shape[1]
    return pl.pallas_call(
        functools.partial(_fc_body, relu=relu),
        out_shape=jax.ShapeDtypeStruct((B, Nout), out_dtype),
        grid=(Nout // tn,),
        in_specs=[
            pl.BlockSpec((B, K), lambda j: (0, 0)),
            pl.BlockSpec((K, tn), lambda j: (0, j)),
            pl.BlockSpec((1, tn), lambda j: (0, j)),
            pl.BlockSpec((B, tn), lambda j: (0, j)),
        ],
        out_specs=pl.BlockSpec((B, tn), lambda j: (0, j)),
        compiler_params=pltpu.CompilerParams(
            dimension_semantics=("parallel",)),
    )(x, w, b.reshape(1, Nout).astype(jnp.float32), scale.astype(jnp.float32))


def kernel(conv0_w, conv0_b, conv1_w, conv1_b, conv2_w, conv2_b, conv3_w,
           conv3_b, conv4_w, conv4_b, conv5_w, conv5_b, conv6_w, conv6_b,
           conv7_w, conv7_b, conv8_w, conv8_b, conv9_w, conv9_b, conv10_w,
           conv10_b, conv11_w, conv11_b, conv12_w, conv12_b, fc1_w, fc1_b,
           fc2_w, fc2_b, fc3_w, fc3_b, x_nhwc, drop_key):
    convs = [(conv0_w, conv0_b), (conv1_w, conv1_b), (conv2_w, conv2_b),
             (conv3_w, conv3_b), (conv4_w, conv4_b), (conv5_w, conv5_b),
             (conv6_w, conv6_b), (conv7_w, conv7_b), (conv8_w, conv8_b),
             (conv9_w, conv9_b), (conv10_w, conv10_b), (conv11_w, conv11_b),
             (conv12_w, conv12_b)]
    N, H, W, _ = x_nhwc.shape

    x = _conv0(x_nhwc, conv0_w, conv0_b)
    x = jnp.pad(x.reshape(N, 224, 226, 64), ((0, 0), (2, 2), (0, 0), (0, 0)))
    x = _conv_banded(_bands(x, 28), conv1_w, conv1_b, H=224, W=224, tr=28,
                     pool=True, group=9)
    x = jnp.pad(x.reshape(N, 112, 114, 64), ((0, 0), (2, 2), (0, 0), (0, 0)))
    x = x.reshape(N, 116 * 114, 64)

    cfg = [
        (112, 112, False, 9),
        (112, 112, True, 9),
        (56, 56, False, 9),
        (56, 56, False, 1),
        (56, 56, True, 1),
        (28, 28, False, 1),
        (28, 28, False, 1),
        (28, 28, True, 1),
        (14, 14, False, 1),
        (14, 14, False, 1),
        (14, 14, True, 1),
    ]
    for li, (h, w_sp, pool, group) in enumerate(cfg):
        wq, bq = convs[li + 2]
        pad_out = li != 10
        x = _conv(x, wq, bq, H=h, W=w_sp, pool=pool, pad_out=pad_out,
                  group=group)

    x = jnp.transpose(x, (0, 2, 1)).reshape(N, 512 * 7 * 7)

    k1, k2 = jax.random.split(drop_key)
    s1 = jax.random.bernoulli(k1, 0.5, (N, 4096)).astype(jnp.float32) / 0.5
    s2 = jax.random.bernoulli(k2, 0.5, (N, 4096)).astype(jnp.float32) / 0.5

    x = _fc(x, fc1_w, fc1_b, s1, relu=True, tn=256, out_dtype=jnp.bfloat16)
    x = _fc(x, fc2_w, fc2_b, s2, relu=True, tn=256, out_dtype=jnp.bfloat16)
    ones = jnp.ones((N, fc3_w.shape[1]), jnp.float32)
    x = _fc(x, fc3_w, fc3_b, ones, relu=False, tn=fc3_w.shape[1],
            out_dtype=jnp.float32)
    return x


# speedup vs baseline: 1.6665x; 1.1133x over previous
import functools

import jax
import jax.numpy as jnp
from jax.experimental import pallas as pl
from jax.experimental.pallas import tpu as pltpu


def _colmask(m2, Wp):
    col = jnp.arange(m2, dtype=jnp.int32) % Wp
    return (jnp.logical_and(col != 0, col != Wp - 1)
            .astype(jnp.float32).reshape(m2, 1))


def _pool2x2(y, *, tr, Wp, tco, pad_out):
    tr2 = tr // 2
    Wh = Wp // 2
    W2 = (Wp - 2) // 2
    v = y.reshape(tr2, 2, Wp, tco)
    h = jnp.maximum(v[:, 0], v[:, 1]).reshape(tr2 * Wp, tco)
    sh = jnp.pad(h[1:], ((0, 1), (0, 0)))
    t = jnp.maximum(h, sh).reshape(tr2 * Wh, 2, tco)
    m = t[:, 1, :].reshape(tr2, Wh, tco)
    if pad_out:
        return jnp.pad(m[:, :W2, :], ((0, 0), (1, 1), (0, 0)))
    return m[:, :W2, :]


def _conv_body(x_ref, w_ref, b_ref, m_ref, o_ref, *, H, W, tr, tco, group,
               pool, pad_out):
    Wp = W + 2
    m2 = tr * Wp
    nt = H // tr
    H2 = H // 2 if pool else H
    W2 = W // 2 if pool else W
    Wp2 = W2 + 2
    if pad_out:
        zrow = jnp.zeros((2 * Wp2, tco), o_ref.dtype)
        o_ref[0, pl.ds(0, 2 * Wp2), :] = zrow
        o_ref[0, pl.ds((H2 + 2) * Wp2, 2 * Wp2), :] = zrow
    for i in range(nt):
        base = i * tr * Wp
        if group == 9:
            lhs = jnp.concatenate(
                [x_ref[0, pl.ds(base + (dy + 1) * Wp + dx - 1, m2), :]
                 for dy in range(3) for dx in range(3)], axis=-1)
            acc = jnp.dot(lhs, w_ref[0], preferred_element_type=jnp.float32)
        elif group == 3:
            acc = jnp.zeros((m2, tco), jnp.float32)
            for dy in range(3):
                lhs = jnp.concatenate(
                    [x_ref[0, pl.ds(base + (dy + 1) * Wp + dx - 1, m2), :]
                     for dx in range(3)], axis=-1)
                acc = acc + jnp.dot(lhs, w_ref[dy],
                                    preferred_element_type=jnp.float32)
        else:
            acc = jnp.zeros((m2, tco), jnp.float32)
            for dy in range(3):
                for dx in range(3):
                    acc = acc + jnp.dot(
                        x_ref[0, pl.ds(base + (dy + 1) * Wp + dx - 1, m2), :],
                        w_ref[dy * 3 + dx],
                        preferred_element_type=jnp.float32)
        y = jnp.maximum(acc + b_ref[...], 0.0)
        if pool:
            tr2 = tr // 2
            pv = _pool2x2(y, tr=tr, Wp=Wp, tco=tco, pad_out=pad_out)
            if pad_out:
                o_ref[0, pl.ds((i * tr2 + 2) * Wp2, tr2 * Wp2), :] = (
                    pv.reshape(tr2 * Wp2, tco).astype(o_ref.dtype))
            else:
                o_ref[0, pl.ds(i * tr2 * W2, tr2 * W2), :] = (
                    pv.reshape(tr2 * W2, tco).astype(o_ref.dtype))
        else:
            yv = y * m_ref[...]
            o_ref[0, pl.ds((i * tr + 2) * Wp, m2), :] = yv.astype(o_ref.dtype)


def _conv0_body(x_ref, w_ref, b_ref, m_ref, o_ref):
    y = jnp.maximum(
        jnp.dot(x_ref[0], w_ref[...], preferred_element_type=jnp.float32)
        + b_ref[...], 0.0)
    o_ref[0] = (y * m_ref[...]).astype(o_ref.dtype)


def _conv0(x_nhwc, w, b):
    N = x_nhwc.shape[0]
    Wp = 226
    xq = jnp.pad(x_nhwc.astype(jnp.bfloat16), ((0, 0), (1, 1), (2, 2), (0, 0)))
    pat = jnp.concatenate(
        [xq[:, dy:dy + 224, dx:dx + 226, :] for dy in range(3)
         for dx in range(3)], axis=-1)
    pat = pat.reshape(N, 224 * 226, 27)
    mrows = 56 * 226
    nr = (224 * 226) // mrows
    return pl.pallas_call(
        _conv0_body,
        out_shape=jax.ShapeDtypeStruct((N, 224 * 226, 64), jnp.bfloat16),
        grid=(N, nr),
        in_specs=[
            pl.BlockSpec((1, mrows, 27), lambda n, r: (n, r, 0)),
            pl.BlockSpec((27, 64), lambda n, r: (0, 0)),
            pl.BlockSpec((1, 64), lambda n, r: (0, 0)),
            pl.BlockSpec((mrows, 1), lambda n, r: (0, 0)),
        ],
        out_specs=pl.BlockSpec((1, mrows, 64), lambda n, r: (n, r, 0)),
        compiler_params=pltpu.CompilerParams(
            dimension_semantics=("parallel", "parallel")),
    )(pat, w.reshape(27, 64).astype(jnp.bfloat16),
      b.reshape(1, 64).astype(jnp.float32), _colmask(mrows, Wp))


def _conv_band_body(x_ref, w_ref, b_ref, o_ref, *, W, tr, tco, group):
    Wp = W + 2
    tc = 28 if tr % 28 == 0 else tr
    m2 = tc * Wp
    W2 = W // 2
    for i in range(tr // tc):
        base = i * tc * Wp
        if group == 9:
            lhs = jnp.concatenate(
                [x_ref[0, 0, pl.ds(base + (dy + 1) * Wp + dx - 1, m2), :]
                 for dy in range(3) for dx in range(3)], axis=-1)
            acc = jnp.dot(lhs, w_ref[0], preferred_element_type=jnp.float32)
        else:
            acc = jnp.zeros((m2, tco), jnp.float32)
            for dy in range(3):
                lhs = jnp.concatenate(
                    [x_ref[0, 0, pl.ds(base + (dy + 1) * Wp + dx - 1, m2), :]
                     for dx in range(3)], axis=-1)
                acc = acc + jnp.dot(lhs, w_ref[dy],
                                    preferred_element_type=jnp.float32)
        y = jnp.maximum(acc + b_ref[...], 0.0)
        tc2 = tc // 2
        pv = _pool2x2(y, tr=tc, Wp=Wp, tco=tco, pad_out=True)
        o_ref[0, 0, pl.ds(i * tc2 * (W2 + 2), tc2 * (W2 + 2)), :] = (
            pv.reshape(tc2 * (W2 + 2), tco).astype(o_ref.dtype))


def _conv_banded(x, w, b, *, H, W, tr, pool, group):
    N, nt = x.shape[0], x.shape[1]
    cin = x.shape[-1]
    cout = w.shape[-1]
    tco = min(cout, 256)
    Wp = W + 2
    if pool:
        orows = (tr // 2) * (W // 2 + 2)
    else:
        orows = tr * Wp
    if group == 9:
        wm = w.reshape(1, 9 * cin, cout).astype(jnp.bfloat16)
        wspec = pl.BlockSpec((1, 9 * cin, tco), lambda n, c: (0, 0, 0))
    else:
        wm = w.reshape(3, 3 * cin, cout).astype(jnp.bfloat16)
        wspec = pl.BlockSpec((3, 3 * cin, tco), lambda n, c: (0, 0, 0))
    out = pl.pallas_call(
        functools.partial(_conv_band_body, W=W, tr=tr, tco=tco, group=group),
        out_shape=jax.ShapeDtypeStruct((N, nt, orows, cout), jnp.bfloat16),
        grid=(N, nt),
        in_specs=[
            pl.BlockSpec((1, 1, (tr + 4) * Wp, cin),
                         lambda n, c: (n, c, 0, 0)),
            wspec,
            pl.BlockSpec((1, tco), lambda n, c: (0, 0)),
        ],
        out_specs=pl.BlockSpec((1, 1, orows, tco), lambda n, c: (n, c, 0, 0)),
        compiler_params=pltpu.CompilerParams(
            dimension_semantics=("parallel", "parallel")),
    )(x, wm, b.reshape(1, cout).astype(jnp.float32))
    return out.reshape(N, nt * orows, cout)


def _bands(xp, tr):
    N, Hp, Wp, C = xp.shape
    nt = (Hp - 4) // tr
    t = jnp.stack([xp[:, c * tr:c * tr + tr + 4] for c in range(nt)], axis=1)
    return t.reshape(N, nt, (tr + 4) * Wp, C)


def _conv(x, w, b, *, H, W, pool, pad_out, group):
    N = x.shape[0]
    cin = x.shape[-1]
    cout = w.shape[-1]
    tco = min(cout, 256)
    if H % 28 == 0:
        tr = 28
    elif H % 14 == 0:
        tr = 14
    else:
        tr = H
    Wp = W + 2
    Hp = H + 4
    H2 = H // 2 if pool else H
    W2 = W // 2 if pool else W
    out_rows = (H2 + 4) * (W2 + 2) if pad_out else H2 * W2
    if group == 9:
        wm = w.reshape(1, 9 * cin, cout).astype(jnp.bfloat16)
        wspec = pl.BlockSpec((1, 9 * cin, tco), lambda n, j: (0, 0, j))
    elif group == 3:
        wm = w.reshape(3, 3 * cin, cout).astype(jnp.bfloat16)
        wspec = pl.BlockSpec((3, 3 * cin, tco), lambda n, j: (0, 0, j))
    else:
        wm = w.reshape(9, cin, cout).astype(jnp.bfloat16)
        wspec = pl.BlockSpec((9, cin, tco), lambda n, j: (0, 0, j))
    return pl.pallas_call(
        functools.partial(_conv_body, H=H, W=W, tr=tr, tco=tco, group=group,
                          pool=pool, pad_out=pad_out),
        out_shape=jax.ShapeDtypeStruct((N, out_rows, cout), jnp.bfloat16),
        grid=(N, cout // tco),
        in_specs=[
            pl.BlockSpec((1, Hp * Wp, cin), lambda n, j: (n, 0, 0)),
            wspec,
            pl.BlockSpec((1, tco), lambda n, j: (0, j)),
            pl.BlockSpec((tr * Wp, 1), lambda n, j: (0, 0)),
        ],
        out_specs=pl.BlockSpec((1, out_rows, tco), lambda n, j: (n, 0, j)),
        compiler_params=pltpu.CompilerParams(
            dimension_semantics=("parallel", "arbitrary")),
    )(x, wm, b.reshape(1, cout).astype(jnp.float32),
      _colmask(tr * Wp, Wp))


def _chain14_body(x_ref, w0_ref, b0_ref, w1_ref, b1_ref, w2_ref, b2_ref,
                  m_ref, o_ref):
    Wp = 16
    m2 = 14 * Wp
    x = x_ref[0]
    for li, (w_ref, b_ref) in enumerate(((w0_ref, b0_ref), (w1_ref, b1_ref),
                                         (w2_ref, b2_ref))):
        acc = jnp.zeros((m2, 512), jnp.float32)
        for dy in range(3):
            for dx in range(3):
                off = (dy + 1) * Wp + dx - 1
                acc = acc + jnp.dot(
                    x[off:off + m2, :],
                    w_ref[dy * 3 + dx], preferred_element_type=jnp.float32)
        y = jnp.maximum(acc + b_ref[...], 0.0)
        if li < 2:
            yv = (y * m_ref[...]).astype(jnp.bfloat16)
            x = jnp.pad(yv, ((2 * Wp, 2 * Wp), (0, 0)))
        else:
            pv = _pool2x2(y, tr=14, Wp=Wp, tco=512, pad_out=False)
            o_ref[0] = pv.reshape(49, 512).astype(o_ref.dtype)


def _chain28_body(x_ref, w0_ref, b0_ref, w1_ref, b1_ref, w2_ref, b2_ref,
                  m_ref, o_ref):
    Wp = 30
    m2 = 28 * Wp
    x = x_ref[0]
    for li, (w_ref, b_ref) in enumerate(((w0_ref, b0_ref), (w1_ref, b1_ref),
                                         (w2_ref, b2_ref))):
        acc = jnp.zeros((m2, 512), jnp.float32)
        for dy in range(3):
            for dx in range(3):
                off = (dy + 1) * Wp + dx - 1
                acc = acc + jnp.dot(
                    x[off:off + m2, :],
                    w_ref[dy * 3 + dx], preferred_element_type=jnp.float32)
        y = jnp.maximum(acc + b_ref[...], 0.0)
        if li < 2:
            yv = (y * m_ref[...]).astype(jnp.bfloat16)
            x = jnp.pad(yv, ((2 * Wp, 2 * Wp), (0, 0)))
        else:
            Wp2 = 16
            zrow = jnp.zeros((2 * Wp2, 512), o_ref.dtype)
            o_ref[0, pl.ds(0, 2 * Wp2), :] = zrow
            o_ref[0, pl.ds(16 * Wp2, 2 * Wp2), :] = zrow
            pv = _pool2x2(y, tr=28, Wp=Wp, tco=512, pad_out=True)
            o_ref[0, pl.ds(2 * Wp2, 14 * Wp2), :] = (
                pv.reshape(14 * Wp2, 512).astype(o_ref.dtype))


def _chain28(x, w7, b7, w8, b8, w9, b9):
    N = x.shape[0]
    args = []
    specs = []
    for w, b in ((w7, b7), (w8, b8), (w9, b9)):
        cin = w.shape[2]
        args.append(w.reshape(9, cin, 512).astype(jnp.bfloat16))
        args.append(b.reshape(1, 512).astype(jnp.float32))
        specs.append(pl.BlockSpec((9, cin, 512), lambda n: (0, 0, 0)))
        specs.append(pl.BlockSpec((1, 512), lambda n: (0, 0)))
    return pl.pallas_call(
        _chain28_body,
        out_shape=jax.ShapeDtypeStruct((N, 18 * 16, 512), jnp.bfloat16),
        grid=(N,),
        in_specs=[pl.BlockSpec((1, 32 * 30, 256), lambda n: (n, 0, 0))]
        + specs + [pl.BlockSpec((28 * 30, 1), lambda n: (0, 0))],
        out_specs=pl.BlockSpec((1, 18 * 16, 512), lambda n: (n, 0, 0)),
        compiler_params=pltpu.CompilerParams(
            dimension_semantics=("parallel",)),
    )(x, *args, _colmask(28 * 30, 30))


def _chain14(x, w10, b10, w11, b11, w12, b12):
    N = x.shape[0]
    args = []
    for w, b in ((w10, b10), (w11, b11), (w12, b12)):
        args.append(w.reshape(9, 512, 512).astype(jnp.bfloat16))
        args.append(b.reshape(1, 512).astype(jnp.float32))
    wspec = pl.BlockSpec((9, 512, 512), lambda n: (0, 0, 0))
    bspec = pl.BlockSpec((1, 512), lambda n: (0, 0))
    return pl.pallas_call(
        _chain14_body,
        out_shape=jax.ShapeDtypeStruct((N, 49, 512), jnp.bfloat16),
        grid=(N,),
        in_specs=[
            pl.BlockSpec((1, 18 * 16, 512), lambda n: (n, 0, 0)),
            wspec, bspec, wspec, bspec, wspec, bspec,
            pl.BlockSpec((14 * 16, 1), lambda n: (0, 0)),
        ],
        out_specs=pl.BlockSpec((1, 49, 512), lambda n: (n, 0, 0)),
        compiler_params=pltpu.CompilerParams(
            dimension_semantics=("parallel",)),
    )(x, *args, _colmask(14 * 16, 16))


def _fc_body(x_ref, w_ref, b_ref, s_ref, o_ref, *, relu):
    y = jnp.dot(x_ref[...], w_ref[...],
                preferred_element_type=jnp.float32) + b_ref[...]
    if relu:
        y = jnp.maximum(y, 0.0)
    o_ref[...] = (y * s_ref[...]).astype(o_ref.dtype)


def _fc(x, w, b, scale, *, relu, tn, out_dtype):
    B, K = x.shape
    Nout = w.shape[1]
    return pl.pallas_call(
        functools.partial(_fc_body, relu=relu),
        out_shape=jax.ShapeDtypeStruct((B, Nout), out_dtype),
        grid=(Nout // tn,),
        in_specs=[
            pl.BlockSpec((B, K), lambda j: (0, 0)),
            pl.BlockSpec((K, tn), lambda j: (0, j)),
            pl.BlockSpec((1, tn), lambda j: (0, j)),
            pl.BlockSpec((B, tn), lambda j: (0, j)),
        ],
        out_specs=pl.BlockSpec((B, tn), lambda j: (0, j)),
        compiler_params=pltpu.CompilerParams(
            dimension_semantics=("parallel",)),
    )(x, w, b.reshape(1, Nout).astype(jnp.float32), scale.astype(jnp.float32))


def kernel(conv0_w, conv0_b, conv1_w, conv1_b, conv2_w, conv2_b, conv3_w,
           conv3_b, conv4_w, conv4_b, conv5_w, conv5_b, conv6_w, conv6_b,
           conv7_w, conv7_b, conv8_w, conv8_b, conv9_w, conv9_b, conv10_w,
           conv10_b, conv11_w, conv11_b, conv12_w, conv12_b, fc1_w, fc1_b,
           fc2_w, fc2_b, fc3_w, fc3_b, x_nhwc, drop_key):
    convs = [(conv0_w, conv0_b), (conv1_w, conv1_b), (conv2_w, conv2_b),
             (conv3_w, conv3_b), (conv4_w, conv4_b), (conv5_w, conv5_b),
             (conv6_w, conv6_b), (conv7_w, conv7_b), (conv8_w, conv8_b),
             (conv9_w, conv9_b), (conv10_w, conv10_b), (conv11_w, conv11_b),
             (conv12_w, conv12_b)]
    N, H, W, _ = x_nhwc.shape

    x = _conv0(x_nhwc, conv0_w, conv0_b)
    x = jnp.pad(x.reshape(N, 224, 226, 64), ((0, 0), (2, 2), (0, 0), (0, 0)))
    x = _conv_banded(_bands(x, 56), conv1_w, conv1_b, H=224, W=224, tr=56,
                     pool=True, group=9)
    x = jnp.pad(x.reshape(N, 112, 114, 64), ((0, 0), (2, 2), (0, 0), (0, 0)))
    x = x.reshape(N, 116 * 114, 64)

    cfg = [
        (112, 112, False, 9),
        (112, 112, True, 9),
        (56, 56, False, 9),
        (56, 56, False, 1),
        (56, 56, True, 1),
    ]
    for li, (h, w_sp, pool, group) in enumerate(cfg):
        wq, bq = convs[li + 2]
        x = _conv(x, wq, bq, H=h, W=w_sp, pool=pool, pad_out=True,
                  group=group)
    x = _chain28(x, conv7_w, conv7_b, conv8_w, conv8_b, conv9_w, conv9_b)
    x = _chain14(x, conv10_w, conv10_b, conv11_w, conv11_b,
                 conv12_w, conv12_b)

    x = jnp.transpose(x, (0, 2, 1)).reshape(N, 512 * 7 * 7)

    k1, k2 = jax.random.split(drop_key)
    s1 = jax.random.bernoulli(k1, 0.5, (N, 4096)).astype(jnp.float32) / 0.5
    s2 = jax.random.bernoulli(k2, 0.5, (N, 4096)).astype(jnp.float32) / 0.5

    x = _fc(x, fc1_w, fc1_b, s1, relu=True, tn=256, out_dtype=jnp.bfloat16)
    x = _fc(x, fc2_w, fc2_b, s2, relu=True, tn=256, out_dtype=jnp.bfloat16)
    ones = jnp.ones((N, fc3_w.shape[1]), jnp.float32)
    x = _fc(x, fc3_w, fc3_b, ones, relu=False, tn=fc3_w.shape[1],
            out_dtype=jnp.float32)
    return x


# speedup vs baseline: 1.6765x; 1.0060x over previous
import functools

import jax
import jax.numpy as jnp
from jax.experimental import pallas as pl
from jax.experimental.pallas import tpu as pltpu


def _colmask(m2, Wp):
    col = jnp.arange(m2, dtype=jnp.int32) % Wp
    return (jnp.logical_and(col != 0, col != Wp - 1)
            .astype(jnp.float32).reshape(m2, 1))


def _pool2x2(y, *, tr, Wp, tco, pad_out):
    tr2 = tr // 2
    Wh = Wp // 2
    W2 = (Wp - 2) // 2
    v = y.reshape(tr2, 2, Wp, tco)
    h = jnp.maximum(v[:, 0], v[:, 1]).reshape(tr2 * Wp, tco)
    sh = jnp.pad(h[1:], ((0, 1), (0, 0)))
    t = jnp.maximum(h, sh).reshape(tr2 * Wh, 2, tco)
    m = t[:, 1, :].reshape(tr2, Wh, tco)
    if pad_out:
        return jnp.pad(m[:, :W2, :], ((0, 0), (1, 1), (0, 0)))
    return m[:, :W2, :]


def _conv_body(x_ref, w_ref, b_ref, m_ref, o_ref, *, H, W, tr, tco, group,
               pool, pad_out):
    Wp = W + 2
    m2 = tr * Wp
    nt = H // tr
    H2 = H // 2 if pool else H
    W2 = W // 2 if pool else W
    Wp2 = W2 + 2
    if pad_out:
        zrow = jnp.zeros((2 * Wp2, tco), o_ref.dtype)
        o_ref[0, pl.ds(0, 2 * Wp2), :] = zrow
        o_ref[0, pl.ds((H2 + 2) * Wp2, 2 * Wp2), :] = zrow
    for i in range(nt):
        base = i * tr * Wp
        if group == 9:
            lhs = jnp.concatenate(
                [x_ref[0, pl.ds(base + (dy + 1) * Wp + dx - 1, m2), :]
                 for dy in range(3) for dx in range(3)], axis=-1)
            acc = jnp.dot(lhs, w_ref[0], preferred_element_type=jnp.float32)
        elif group == 3:
            acc = jnp.zeros((m2, tco), jnp.float32)
            for dy in range(3):
                lhs = jnp.concatenate(
                    [x_ref[0, pl.ds(base + (dy + 1) * Wp + dx - 1, m2), :]
                     for dx in range(3)], axis=-1)
                acc = acc + jnp.dot(lhs, w_ref[dy],
                                    preferred_element_type=jnp.float32)
        else:
            acc = jnp.zeros((m2, tco), jnp.float32)
            for dy in range(3):
                for dx in range(3):
                    acc = acc + jnp.dot(
                        x_ref[0, pl.ds(base + (dy + 1) * Wp + dx - 1, m2), :],
                        w_ref[dy * 3 + dx],
                        preferred_element_type=jnp.float32)
        y = jnp.maximum(acc + b_ref[...], 0.0)
        if pool:
            tr2 = tr // 2
            pv = _pool2x2(y, tr=tr, Wp=Wp, tco=tco, pad_out=pad_out)
            if pad_out:
                o_ref[0, pl.ds((i * tr2 + 2) * Wp2, tr2 * Wp2), :] = (
                    pv.reshape(tr2 * Wp2, tco).astype(o_ref.dtype))
            else:
                o_ref[0, pl.ds(i * tr2 * W2, tr2 * W2), :] = (
                    pv.reshape(tr2 * W2, tco).astype(o_ref.dtype))
        else:
            yv = y * m_ref[...]
            o_ref[0, pl.ds((i * tr + 2) * Wp, m2), :] = yv.astype(o_ref.dtype)


def _conv0_body(x_ref, w_ref, b_ref, m_ref, o_ref):
    y = jnp.maximum(
        jnp.dot(x_ref[0], w_ref[...], preferred_element_type=jnp.float32)
        + b_ref[...], 0.0)
    o_ref[0] = (y * m_ref[...]).astype(o_ref.dtype)


def _conv0(x_nhwc, w, b):
    N = x_nhwc.shape[0]
    Wp = 226
    xq = jnp.pad(x_nhwc.astype(jnp.bfloat16), ((0, 0), (1, 1), (2, 2), (0, 0)))
    pat = jnp.concatenate(
        [xq[:, dy:dy + 224, dx:dx + 226, :] for dy in range(3)
         for dx in range(3)], axis=-1)
    pat = pat.reshape(N, 224 * 226, 27)
    mrows = 56 * 226
    nr = (224 * 226) // mrows
    return pl.pallas_call(
        _conv0_body,
        out_shape=jax.ShapeDtypeStruct((N, 224 * 226, 64), jnp.bfloat16),
        grid=(N, nr),
        in_specs=[
            pl.BlockSpec((1, mrows, 27), lambda n, r: (n, r, 0)),
            pl.BlockSpec((27, 64), lambda n, r: (0, 0)),
            pl.BlockSpec((1, 64), lambda n, r: (0, 0)),
            pl.BlockSpec((mrows, 1), lambda n, r: (0, 0)),
        ],
        out_specs=pl.BlockSpec((1, mrows, 64), lambda n, r: (n, r, 0)),
        compiler_params=pltpu.CompilerParams(
            dimension_semantics=("parallel", "parallel")),
    )(pat, w.reshape(27, 64).astype(jnp.bfloat16),
      b.reshape(1, 64).astype(jnp.float32), _colmask(mrows, Wp))


def _conv_band_body(x_ref, w_ref, b_ref, o_ref, *, W, tr, tco, group):
    Wp = W + 2
    tc = 28 if tr % 28 == 0 else tr
    m2 = tc * Wp
    W2 = W // 2
    for i in range(tr // tc):
        base = i * tc * Wp
        if group == 9:
            lhs = jnp.concatenate(
                [x_ref[0, 0, pl.ds(base + (dy + 1) * Wp + dx - 1, m2), :]
                 for dy in range(3) for dx in range(3)], axis=-1)
            acc = jnp.dot(lhs, w_ref[0], preferred_element_type=jnp.float32)
        else:
            acc = jnp.zeros((m2, tco), jnp.float32)
            for dy in range(3):
                lhs = jnp.concatenate(
                    [x_ref[0, 0, pl.ds(base + (dy + 1) * Wp + dx - 1, m2), :]
                     for dx in range(3)], axis=-1)
                acc = acc + jnp.dot(lhs, w_ref[dy],
                                    preferred_element_type=jnp.float32)
        y = jnp.maximum(acc + b_ref[...], 0.0)
        tc2 = tc // 2
        pv = _pool2x2(y, tr=tc, Wp=Wp, tco=tco, pad_out=True)
        o_ref[0, 0, pl.ds(i * tc2 * (W2 + 2), tc2 * (W2 + 2)), :] = (
            pv.reshape(tc2 * (W2 + 2), tco).astype(o_ref.dtype))


def _conv_banded(x, w, b, *, H, W, tr, pool, group):
    N, nt = x.shape[0], x.shape[1]
    cin = x.shape[-1]
    cout = w.shape[-1]
    tco = min(cout, 256)
    Wp = W + 2
    if pool:
        orows = (tr // 2) * (W // 2 + 2)
    else:
        orows = tr * Wp
    if group == 9:
        wm = w.reshape(1, 9 * cin, cout).astype(jnp.bfloat16)
        wspec = pl.BlockSpec((1, 9 * cin, tco), lambda n, c: (0, 0, 0))
    else:
        wm = w.reshape(3, 3 * cin, cout).astype(jnp.bfloat16)
        wspec = pl.BlockSpec((3, 3 * cin, tco), lambda n, c: (0, 0, 0))
    out = pl.pallas_call(
        functools.partial(_conv_band_body, W=W, tr=tr, tco=tco, group=group),
        out_shape=jax.ShapeDtypeStruct((N, nt, orows, cout), jnp.bfloat16),
        grid=(N, nt),
        in_specs=[
            pl.BlockSpec((1, 1, (tr + 4) * Wp, cin),
                         lambda n, c: (n, c, 0, 0)),
            wspec,
            pl.BlockSpec((1, tco), lambda n, c: (0, 0)),
        ],
        out_specs=pl.BlockSpec((1, 1, orows, tco), lambda n, c: (n, c, 0, 0)),
        compiler_params=pltpu.CompilerParams(
            dimension_semantics=("parallel", "parallel")),
    )(x, wm, b.reshape(1, cout).astype(jnp.float32))
    return out.reshape(N, nt * orows, cout)


def _bands(xp, tr):
    N, Hp, Wp, C = xp.shape
    nt = (Hp - 4) // tr
    t = jnp.stack([xp[:, c * tr:c * tr + tr + 4] for c in range(nt)], axis=1)
    return t.reshape(N, nt, (tr + 4) * Wp, C)


def _conv(x, w, b, *, H, W, pool, pad_out, group):
    N = x.shape[0]
    cin = x.shape[-1]
    cout = w.shape[-1]
    tco = min(cout, 256)
    if H % 28 == 0:
        tr = 28
    elif H % 14 == 0:
        tr = 14
    else:
        tr = H
    Wp = W + 2
    Hp = H + 4
    H2 = H // 2 if pool else H
    W2 = W // 2 if pool else W
    out_rows = (H2 + 4) * (W2 + 2) if pad_out else H2 * W2
    if group == 9:
        wm = w.reshape(1, 9 * cin, cout).astype(jnp.bfloat16)
        wspec = pl.BlockSpec((1, 9 * cin, tco), lambda n, j: (0, 0, j))
    elif group == 3:
        wm = w.reshape(3, 3 * cin, cout).astype(jnp.bfloat16)
        wspec = pl.BlockSpec((3, 3 * cin, tco), lambda n, j: (0, 0, j))
    else:
        wm = w.reshape(9, cin, cout).astype(jnp.bfloat16)
        wspec = pl.BlockSpec((9, cin, tco), lambda n, j: (0, 0, j))
    return pl.pallas_call(
        functools.partial(_conv_body, H=H, W=W, tr=tr, tco=tco, group=group,
                          pool=pool, pad_out=pad_out),
        out_shape=jax.ShapeDtypeStruct((N, out_rows, cout), jnp.bfloat16),
        grid=(N, cout // tco),
        in_specs=[
            pl.BlockSpec((1, Hp * Wp, cin), lambda n, j: (n, 0, 0)),
            wspec,
            pl.BlockSpec((1, tco), lambda n, j: (0, j)),
            pl.BlockSpec((tr * Wp, 1), lambda n, j: (0, 0)),
        ],
        out_specs=pl.BlockSpec((1, out_rows, tco), lambda n, j: (n, 0, j)),
        compiler_params=pltpu.CompilerParams(
            dimension_semantics=("parallel", "arbitrary")),
    )(x, wm, b.reshape(1, cout).astype(jnp.float32),
      _colmask(tr * Wp, Wp))


def _chain14_body(x_ref, w0_ref, b0_ref, w1_ref, b1_ref, w2_ref, b2_ref,
                  m_ref, o_ref):
    Wp = 16
    m2 = 14 * Wp
    x = x_ref[0]
    for li, (w_ref, b_ref) in enumerate(((w0_ref, b0_ref), (w1_ref, b1_ref),
                                         (w2_ref, b2_ref))):
        acc = jnp.zeros((m2, 512), jnp.float32)
        for dy in range(3):
            for dx in range(3):
                off = (dy + 1) * Wp + dx - 1
                acc = acc + jnp.dot(
                    x[off:off + m2, :],
                    w_ref[dy * 3 + dx], preferred_element_type=jnp.float32)
        y = jnp.maximum(acc + b_ref[...], 0.0)
        if li < 2:
            yv = (y * m_ref[...]).astype(jnp.bfloat16)
            x = jnp.pad(yv, ((2 * Wp, 2 * Wp), (0, 0)))
        else:
            pv = _pool2x2(y, tr=14, Wp=Wp, tco=512, pad_out=False)
            o_ref[0] = pv.reshape(49, 512).astype(o_ref.dtype)


def _chain56_body(x_ref, w0_ref, b0_ref, w1_ref, b1_ref, w2_ref, b2_ref,
                  m_ref, o_ref, s1_ref, s2_ref):
    Wp = 58
    tr = 28
    m2 = tr * Wp
    H = 56
    for li in range(3):
        if li == 0:
            src, w_ref, b_ref, dst = x_ref, w0_ref, b0_ref, s1_ref
        elif li == 1:
            src, w_ref, b_ref, dst = s1_ref, w1_ref, b1_ref, s2_ref
        else:
            src, w_ref, b_ref, dst = s2_ref, w2_ref, b2_ref, None
        if dst is not None:
            zrow = jnp.zeros((2 * Wp, 256), jnp.bfloat16)
            dst[pl.ds(0, 2 * Wp), :] = zrow
            dst[pl.ds((H + 2) * Wp, 2 * Wp), :] = zrow
        else:
            Wp2 = 30
            zr2 = jnp.zeros((2 * Wp2, 256), o_ref.dtype)
            o_ref[0, pl.ds(0, 2 * Wp2), :] = zr2
            o_ref[0, pl.ds(30 * Wp2, 2 * Wp2), :] = zr2
        for i in range(2):
            base = i * m2

            def _ld(off):
                if li == 0:
                    return x_ref[0, pl.ds(base + off, m2), :]
                return src[pl.ds(base + off, m2), :]

            if li == 0:
                lhs = jnp.concatenate(
                    [_ld((dy + 1) * Wp + dx - 1)
                     for dy in range(3) for dx in range(3)], axis=-1)
                acc = jnp.dot(lhs, w_ref[0],
                              preferred_element_type=jnp.float32)
            else:
                acc = jnp.zeros((m2, 256), jnp.float32)
                for dy in range(3):
                    for dx in range(3):
                        acc = acc + jnp.dot(
                            _ld((dy + 1) * Wp + dx - 1),
                            w_ref[dy * 3 + dx],
                            preferred_element_type=jnp.float32)
            y = jnp.maximum(acc + b_ref[...], 0.0)
            if dst is not None:
                yv = (y * m_ref[...]).astype(jnp.bfloat16)
                dst[pl.ds((i * tr + 2) * Wp, m2), :] = yv
            else:
                pv = _pool2x2(y, tr=tr, Wp=Wp, tco=256, pad_out=True)
                o_ref[0, pl.ds((i * 14 + 2) * 30, 14 * 30), :] = (
                    pv.reshape(14 * 30, 256).astype(o_ref.dtype))


def _chain56(x, w4, b4, w5, b5, w6, b6):
    N = x.shape[0]
    wm4 = w4.reshape(1, 9 * 128, 256).astype(jnp.bfloat16)
    args = [wm4, b4.reshape(1, 256).astype(jnp.float32)]
    specs = [pl.BlockSpec((1, 9 * 128, 256), lambda n: (0, 0, 0)),
             pl.BlockSpec((1, 256), lambda n: (0, 0))]
    for w, b in ((w5, b5), (w6, b6)):
        args.append(w.reshape(9, 256, 256).astype(jnp.bfloat16))
        args.append(b.reshape(1, 256).astype(jnp.float32))
        specs.append(pl.BlockSpec((9, 256, 256), lambda n: (0, 0, 0)))
        specs.append(pl.BlockSpec((1, 256), lambda n: (0, 0)))
    grid_spec = pltpu.PrefetchScalarGridSpec(
        num_scalar_prefetch=0,
        grid=(N,),
        in_specs=[pl.BlockSpec((1, 60 * 58, 128), lambda n: (n, 0, 0))]
        + specs + [pl.BlockSpec((28 * 58, 1), lambda n: (0, 0))],
        out_specs=pl.BlockSpec((1, 32 * 30, 256), lambda n: (n, 0, 0)),
        scratch_shapes=[pltpu.VMEM((60 * 58, 256), jnp.bfloat16),
                        pltpu.VMEM((60 * 58, 256), jnp.bfloat16)],
    )
    return pl.pallas_call(
        _chain56_body,
        out_shape=jax.ShapeDtypeStruct((N, 32 * 30, 256), jnp.bfloat16),
        grid_spec=grid_spec,
        compiler_params=pltpu.CompilerParams(
            dimension_semantics=("parallel",)),
    )(x, *args, _colmask(28 * 58, 58))


def _chain28_body(x_ref, w0_ref, b0_ref, w1_ref, b1_ref, w2_ref, b2_ref,
                  m_ref, o_ref):
    Wp = 30
    m2 = 28 * Wp
    x = x_ref[0]
    for li, (w_ref, b_ref) in enumerate(((w0_ref, b0_ref), (w1_ref, b1_ref),
                                         (w2_ref, b2_ref))):
        acc = jnp.zeros((m2, 512), jnp.float32)
        for dy in range(3):
            for dx in range(3):
                off = (dy + 1) * Wp + dx - 1
                acc = acc + jnp.dot(
                    x[off:off + m2, :],
                    w_ref[dy * 3 + dx], preferred_element_type=jnp.float32)
        y = jnp.maximum(acc + b_ref[...], 0.0)
        if li < 2:
            yv = (y * m_ref[...]).astype(jnp.bfloat16)
            x = jnp.pad(yv, ((2 * Wp, 2 * Wp), (0, 0)))
        else:
            Wp2 = 16
            zrow = jnp.zeros((2 * Wp2, 512), o_ref.dtype)
            o_ref[0, pl.ds(0, 2 * Wp2), :] = zrow
            o_ref[0, pl.ds(16 * Wp2, 2 * Wp2), :] = zrow
            pv = _pool2x2(y, tr=28, Wp=Wp, tco=512, pad_out=True)
            o_ref[0, pl.ds(2 * Wp2, 14 * Wp2), :] = (
                pv.reshape(14 * Wp2, 512).astype(o_ref.dtype))


def _chain28(x, w7, b7, w8, b8, w9, b9):
    N = x.shape[0]
    args = []
    specs = []
    for w, b in ((w7, b7), (w8, b8), (w9, b9)):
        cin = w.shape[2]
        args.append(w.reshape(9, cin, 512).astype(jnp.bfloat16))
        args.append(b.reshape(1, 512).astype(jnp.float32))
        specs.append(pl.BlockSpec((9, cin, 512), lambda n: (0, 0, 0)))
        specs.append(pl.BlockSpec((1, 512), lambda n: (0, 0)))
    return pl.pallas_call(
        _chain28_body,
        out_shape=jax.ShapeDtypeStruct((N, 18 * 16, 512), jnp.bfloat16),
        grid=(N,),
        in_specs=[pl.BlockSpec((1, 32 * 30, 256), lambda n: (n, 0, 0))]
        + specs + [pl.BlockSpec((28 * 30, 1), lambda n: (0, 0))],
        out_specs=pl.BlockSpec((1, 18 * 16, 512), lambda n: (n, 0, 0)),
        compiler_params=pltpu.CompilerParams(
            dimension_semantics=("parallel",)),
    )(x, *args, _colmask(28 * 30, 30))


def _chain14(x, w10, b10, w11, b11, w12, b12):
    N = x.shape[0]
    args = []
    for w, b in ((w10, b10), (w11, b11), (w12, b12)):
        args.append(w.reshape(9, 512, 512).astype(jnp.bfloat16))
        args.append(b.reshape(1, 512).astype(jnp.float32))
    wspec = pl.BlockSpec((9, 512, 512), lambda n: (0, 0, 0))
    bspec = pl.BlockSpec((1, 512), lambda n: (0, 0))
    return pl.pallas_call(
        _chain14_body,
        out_shape=jax.ShapeDtypeStruct((N, 49, 512), jnp.bfloat16),
        grid=(N,),
        in_specs=[
            pl.BlockSpec((1, 18 * 16, 512), lambda n: (n, 0, 0)),
            wspec, bspec, wspec, bspec, wspec, bspec,
            pl.BlockSpec((14 * 16, 1), lambda n: (0, 0)),
        ],
        out_specs=pl.BlockSpec((1, 49, 512), lambda n: (n, 0, 0)),
        compiler_params=pltpu.CompilerParams(
            dimension_semantics=("parallel",)),
    )(x, *args, _colmask(14 * 16, 16))


def _fc_body(x_ref, w_ref, b_ref, s_ref, o_ref, *, relu):
    y = jnp.dot(x_ref[...], w_ref[...],
                preferred_element_type=jnp.float32) + b_ref[...]
    if relu:
        y = jnp.maximum(y, 0.0)
    o_ref[...] = (y * s_ref[...]).astype(o_ref.dtype)


def _fc(x, w, b, scale, *, relu, tn, out_dtype):
    B, K = x.shape
    Nout = w.shape[1]
    return pl.pallas_call(
        functools.partial(_fc_body, relu=relu),
        out_shape=jax.ShapeDtypeStruct((B, Nout), out_dtype),
        grid=(Nout // tn,),
        in_specs=[
            pl.BlockSpec((B, K), lambda j: (0, 0)),
            pl.BlockSpec((K, tn), lambda j: (0, j)),
            pl.BlockSpec((1, tn), lambda j: (0, j)),
            pl.BlockSpec((B, tn), lambda j: (0, j)),
        ],
        out_specs=pl.BlockSpec((B, tn), lambda j: (0, j)),
        compiler_params=pltpu.CompilerParams(
            dimension_semantics=("parallel",)),
    )(x, w, b.reshape(1, Nout).astype(jnp.float32), scale.astype(jnp.float32))


def kernel(conv0_w, conv0_b, conv1_w, conv1_b, conv2_w, conv2_b, conv3_w,
           conv3_b, conv4_w, conv4_b, conv5_w, conv5_b, conv6_w, conv6_b,
           conv7_w, conv7_b, conv8_w, conv8_b, conv9_w, conv9_b, conv10_w,
           conv10_b, conv11_w, conv11_b, conv12_w, conv12_b, fc1_w, fc1_b,
           fc2_w, fc2_b, fc3_w, fc3_b, x_nhwc, drop_key):
    convs = [(conv0_w, conv0_b), (conv1_w, conv1_b), (conv2_w, conv2_b),
             (conv3_w, conv3_b), (conv4_w, conv4_b), (conv5_w, conv5_b),
             (conv6_w, conv6_b), (conv7_w, conv7_b), (conv8_w, conv8_b),
             (conv9_w, conv9_b), (conv10_w, conv10_b), (conv11_w, conv11_b),
             (conv12_w, conv12_b)]
    N, H, W, _ = x_nhwc.shape

    x = _conv0(x_nhwc, conv0_w, conv0_b)
    x = jnp.pad(x.reshape(N, 224, 226, 64), ((0, 0), (2, 2), (0, 0), (0, 0)))
    x = _conv_banded(_bands(x, 56), conv1_w, conv1_b, H=224, W=224, tr=56,
                     pool=True, group=9)
    x = jnp.pad(x.reshape(N, 112, 114, 64), ((0, 0), (2, 2), (0, 0), (0, 0)))
    x = x.reshape(N, 116 * 114, 64)

    cfg = [
        (112, 112, False, 9),
        (112, 112, True, 9),
    ]
    for li, (h, w_sp, pool, group) in enumerate(cfg):
        wq, bq = convs[li + 2]
        x = _conv(x, wq, bq, H=h, W=w_sp, pool=pool, pad_out=True,
                  group=group)
    x = _chain56(x, conv4_w, conv4_b, conv5_w, conv5_b, conv6_w, conv6_b)
    x = _chain28(x, conv7_w, conv7_b, conv8_w, conv8_b, conv9_w, conv9_b)
    x = _chain14(x, conv10_w, conv10_b, conv11_w, conv11_b,
                 conv12_w, conv12_b)

    x = jnp.transpose(x, (0, 2, 1)).reshape(N, 512 * 7 * 7)

    k1, k2 = jax.random.split(drop_key)
    s1 = jax.random.bernoulli(k1, 0.5, (N, 4096)).astype(jnp.float32) / 0.5
    s2 = jax.random.bernoulli(k2, 0.5, (N, 4096)).astype(jnp.float32) / 0.5

    x = _fc(x, fc1_w, fc1_b, s1, relu=True, tn=256, out_dtype=jnp.bfloat16)
    x = _fc(x, fc2_w, fc2_b, s2, relu=True, tn=256, out_dtype=jnp.bfloat16)
    ones = jnp.ones((N, fc3_w.shape[1]), jnp.float32)
    x = _fc(x, fc3_w, fc3_b, ones, relu=False, tn=fc3_w.shape[1],
            out_dtype=jnp.float32)
    return x


# speedup vs baseline: 1.6983x; 1.0130x over previous
import functools

import jax
import jax.numpy as jnp
from jax.experimental import pallas as pl
from jax.experimental.pallas import tpu as pltpu


def _colmask(m2, Wp):
    col = jnp.arange(m2, dtype=jnp.int32) % Wp
    return (jnp.logical_and(col != 0, col != Wp - 1)
            .astype(jnp.float32).reshape(m2, 1))


def _pool2x2(y, *, tr, Wp, tco, pad_out):
    tr2 = tr // 2
    Wh = Wp // 2
    W2 = (Wp - 2) // 2
    v = y.reshape(tr2, 2, Wp, tco)
    h = jnp.maximum(v[:, 0], v[:, 1]).reshape(tr2 * Wp, tco)
    sh = jnp.pad(h[1:], ((0, 1), (0, 0)))
    t = jnp.maximum(h, sh).reshape(tr2 * Wh, 2, tco)
    m = t[:, 1, :].reshape(tr2, Wh, tco)
    if pad_out:
        return jnp.pad(m[:, :W2, :], ((0, 0), (1, 1), (0, 0)))
    return m[:, :W2, :]


def _conv_body(x_ref, w_ref, b_ref, m_ref, o_ref, *, H, W, tr, tco, group,
               pool, pad_out):
    Wp = W + 2
    m2 = tr * Wp
    nt = H // tr
    H2 = H // 2 if pool else H
    W2 = W // 2 if pool else W
    Wp2 = W2 + 2
    if pad_out:
        zrow = jnp.zeros((2 * Wp2, tco), o_ref.dtype)
        o_ref[0, pl.ds(0, 2 * Wp2), :] = zrow
        o_ref[0, pl.ds((H2 + 2) * Wp2, 2 * Wp2), :] = zrow
    for i in range(nt):
        base = i * tr * Wp
        if group == 9:
            lhs = jnp.concatenate(
                [x_ref[0, pl.ds(base + (dy + 1) * Wp + dx - 1, m2), :]
                 for dy in range(3) for dx in range(3)], axis=-1)
            acc = jnp.dot(lhs, w_ref[0], preferred_element_type=jnp.float32)
        elif group == 3:
            acc = jnp.zeros((m2, tco), jnp.float32)
            for dy in range(3):
                lhs = jnp.concatenate(
                    [x_ref[0, pl.ds(base + (dy + 1) * Wp + dx - 1, m2), :]
                     for dx in range(3)], axis=-1)
                acc = acc + jnp.dot(lhs, w_ref[dy],
                                    preferred_element_type=jnp.float32)
        else:
            acc = jnp.zeros((m2, tco), jnp.float32)
            for dy in range(3):
                for dx in range(3):
                    acc = acc + jnp.dot(
                        x_ref[0, pl.ds(base + (dy + 1) * Wp + dx - 1, m2), :],
                        w_ref[dy * 3 + dx],
                        preferred_element_type=jnp.float32)
        y = jnp.maximum(acc + b_ref[...], 0.0)
        if pool:
            tr2 = tr // 2
            pv = _pool2x2(y, tr=tr, Wp=Wp, tco=tco, pad_out=pad_out)
            if pad_out:
                o_ref[0, pl.ds((i * tr2 + 2) * Wp2, tr2 * Wp2), :] = (
                    pv.reshape(tr2 * Wp2, tco).astype(o_ref.dtype))
            else:
                o_ref[0, pl.ds(i * tr2 * W2, tr2 * W2), :] = (
                    pv.reshape(tr2 * W2, tco).astype(o_ref.dtype))
        else:
            yv = y * m_ref[...]
            o_ref[0, pl.ds((i * tr + 2) * Wp, m2), :] = yv.astype(o_ref.dtype)


def _conv0_body(x_ref, w_ref, b_ref, m_ref, o_ref):
    y = jnp.maximum(
        jnp.dot(x_ref[0], w_ref[...], preferred_element_type=jnp.float32)
        + b_ref[...], 0.0)
    o_ref[0] = (y * m_ref[...]).astype(o_ref.dtype)


def _conv0(x_nhwc, w, b):
    N = x_nhwc.shape[0]
    Wp = 226
    xq = jnp.pad(x_nhwc.astype(jnp.bfloat16), ((0, 0), (1, 1), (2, 2), (0, 0)))
    pat = jnp.concatenate(
        [xq[:, dy:dy + 224, dx:dx + 226, :] for dy in range(3)
         for dx in range(3)], axis=-1)
    pat = pat.reshape(N, 224 * 226, 27)
    mrows = 56 * 226
    nr = (224 * 226) // mrows
    return pl.pallas_call(
        _conv0_body,
        out_shape=jax.ShapeDtypeStruct((N, 224 * 226, 64), jnp.bfloat16),
        grid=(N, nr),
        in_specs=[
            pl.BlockSpec((1, mrows, 27), lambda n, r: (n, r, 0)),
            pl.BlockSpec((27, 64), lambda n, r: (0, 0)),
            pl.BlockSpec((1, 64), lambda n, r: (0, 0)),
            pl.BlockSpec((mrows, 1), lambda n, r: (0, 0)),
        ],
        out_specs=pl.BlockSpec((1, mrows, 64), lambda n, r: (n, r, 0)),
        compiler_params=pltpu.CompilerParams(
            dimension_semantics=("parallel", "parallel")),
    )(pat, w.reshape(27, 64).astype(jnp.bfloat16),
      b.reshape(1, 64).astype(jnp.float32), _colmask(mrows, Wp))


def _conv_band_body(x_ref, w_ref, b_ref, o_ref, *, W, tr, tco, group):
    Wp = W + 2
    tc = 28 if tr % 28 == 0 else tr
    m2 = tc * Wp
    W2 = W // 2
    for i in range(tr // tc):
        base = i * tc * Wp
        if group == 9:
            lhs = jnp.concatenate(
                [x_ref[0, 0, pl.ds(base + (dy + 1) * Wp + dx - 1, m2), :]
                 for dy in range(3) for dx in range(3)], axis=-1)
            acc = jnp.dot(lhs, w_ref[0], preferred_element_type=jnp.float32)
        else:
            acc = jnp.zeros((m2, tco), jnp.float32)
            for dy in range(3):
                lhs = jnp.concatenate(
                    [x_ref[0, 0, pl.ds(base + (dy + 1) * Wp + dx - 1, m2), :]
                     for dx in range(3)], axis=-1)
                acc = acc + jnp.dot(lhs, w_ref[dy],
                                    preferred_element_type=jnp.float32)
        y = jnp.maximum(acc + b_ref[...], 0.0)
        tc2 = tc // 2
        pv = _pool2x2(y, tr=tc, Wp=Wp, tco=tco, pad_out=True)
        o_ref[0, 0, pl.ds(i * tc2 * (W2 + 2), tc2 * (W2 + 2)), :] = (
            pv.reshape(tc2 * (W2 + 2), tco).astype(o_ref.dtype))


def _conv_banded(x, w, b, *, H, W, tr, pool, group):
    N, nt = x.shape[0], x.shape[1]
    cin = x.shape[-1]
    cout = w.shape[-1]
    tco = min(cout, 256)
    Wp = W + 2
    if pool:
        orows = (tr // 2) * (W // 2 + 2)
    else:
        orows = tr * Wp
    if group == 9:
        wm = w.reshape(1, 9 * cin, cout).astype(jnp.bfloat16)
        wspec = pl.BlockSpec((1, 9 * cin, tco), lambda n, c: (0, 0, 0))
    else:
        wm = w.reshape(3, 3 * cin, cout).astype(jnp.bfloat16)
        wspec = pl.BlockSpec((3, 3 * cin, tco), lambda n, c: (0, 0, 0))
    out = pl.pallas_call(
        functools.partial(_conv_band_body, W=W, tr=tr, tco=tco, group=group),
        out_shape=jax.ShapeDtypeStruct((N, nt, orows, cout), jnp.bfloat16),
        grid=(N, nt),
        in_specs=[
            pl.BlockSpec((1, 1, (tr + 4) * Wp, cin),
                         lambda n, c: (n, c, 0, 0)),
            wspec,
            pl.BlockSpec((1, tco), lambda n, c: (0, 0)),
        ],
        out_specs=pl.BlockSpec((1, 1, orows, tco), lambda n, c: (n, c, 0, 0)),
        compiler_params=pltpu.CompilerParams(
            dimension_semantics=("parallel", "parallel")),
    )(x, wm, b.reshape(1, cout).astype(jnp.float32))
    return out.reshape(N, nt * orows, cout)


def _bands(xp, tr):
    N, Hp, Wp, C = xp.shape
    nt = (Hp - 4) // tr
    t = jnp.stack([xp[:, c * tr:c * tr + tr + 4] for c in range(nt)], axis=1)
    return t.reshape(N, nt, (tr + 4) * Wp, C)


def _conv(x, w, b, *, H, W, pool, pad_out, group):
    N = x.shape[0]
    cin = x.shape[-1]
    cout = w.shape[-1]
    tco = min(cout, 256)
    if H % 28 == 0:
        tr = 28
    elif H % 14 == 0:
        tr = 14
    else:
        tr = H
    Wp = W + 2
    Hp = H + 4
    H2 = H // 2 if pool else H
    W2 = W // 2 if pool else W
    out_rows = (H2 + 4) * (W2 + 2) if pad_out else H2 * W2
    if group == 9:
        wm = w.reshape(1, 9 * cin, cout).astype(jnp.bfloat16)
        wspec = pl.BlockSpec((1, 9 * cin, tco), lambda n, j: (0, 0, j))
    elif group == 3:
        wm = w.reshape(3, 3 * cin, cout).astype(jnp.bfloat16)
        wspec = pl.BlockSpec((3, 3 * cin, tco), lambda n, j: (0, 0, j))
    else:
        wm = w.reshape(9, cin, cout).astype(jnp.bfloat16)
        wspec = pl.BlockSpec((9, cin, tco), lambda n, j: (0, 0, j))
    return pl.pallas_call(
        functools.partial(_conv_body, H=H, W=W, tr=tr, tco=tco, group=group,
                          pool=pool, pad_out=pad_out),
        out_shape=jax.ShapeDtypeStruct((N, out_rows, cout), jnp.bfloat16),
        grid=(N, cout // tco),
        in_specs=[
            pl.BlockSpec((1, Hp * Wp, cin), lambda n, j: (n, 0, 0)),
            wspec,
            pl.BlockSpec((1, tco), lambda n, j: (0, j)),
            pl.BlockSpec((tr * Wp, 1), lambda n, j: (0, 0)),
        ],
        out_specs=pl.BlockSpec((1, out_rows, tco), lambda n, j: (n, 0, j)),
        compiler_params=pltpu.CompilerParams(
            dimension_semantics=("parallel", "arbitrary")),
    )(x, wm, b.reshape(1, cout).astype(jnp.float32),
      _colmask(tr * Wp, Wp))


def _chain14_body(x_ref, w0_ref, b0_ref, w1_ref, b1_ref, w2_ref, b2_ref,
                  m_ref, o_ref):
    Wp = 16
    m2 = 14 * Wp
    x = x_ref[0]
    for li, (w_ref, b_ref) in enumerate(((w0_ref, b0_ref), (w1_ref, b1_ref),
                                         (w2_ref, b2_ref))):
        acc = jnp.zeros((m2, 512), jnp.float32)
        for dy in range(3):
            for dx in range(3):
                off = (dy + 1) * Wp + dx - 1
                acc = acc + jnp.dot(
                    x[off:off + m2, :],
                    w_ref[dy * 3 + dx], preferred_element_type=jnp.float32)
        y = jnp.maximum(acc + b_ref[...], 0.0)
        if li < 2:
            yv = (y * m_ref[...]).astype(jnp.bfloat16)
            x = jnp.pad(yv, ((2 * Wp, 2 * Wp), (0, 0)))
        else:
            pv = _pool2x2(y, tr=14, Wp=Wp, tco=512, pad_out=False)
            o_ref[0] = pv.reshape(49, 512).astype(o_ref.dtype)


def _chain112_body(x_ref, w0_ref, b0_ref, w1_ref, b1_ref, m_ref, o_ref,
                   s1_ref):
    Wp = 114
    tr = 28
    m2 = tr * Wp
    H = 112
    for li in range(2):
        if li == 0:
            w_ref, b_ref = w0_ref, b0_ref
            zrow = jnp.zeros((2 * Wp, 128), jnp.bfloat16)
            s1_ref[pl.ds(0, 2 * Wp), :] = zrow
            s1_ref[pl.ds((H + 2) * Wp, 2 * Wp), :] = zrow
        else:
            w_ref, b_ref = w1_ref, b1_ref
            Wp2 = 58
            zr2 = jnp.zeros((2 * Wp2, 128), o_ref.dtype)
            o_ref[0, pl.ds(0, 2 * Wp2), :] = zr2
            o_ref[0, pl.ds(58 * Wp2, 2 * Wp2), :] = zr2
        for i in range(4):
            base = i * m2

            def _ld(off):
                if li == 0:
                    return x_ref[0, pl.ds(base + off, m2), :]
                return s1_ref[pl.ds(base + off, m2), :]

            lhs = jnp.concatenate(
                [_ld((dy + 1) * Wp + dx - 1)
                 for dy in range(3) for dx in range(3)], axis=-1)
            acc = jnp.dot(lhs, w_ref[0], preferred_element_type=jnp.float32)
            y = jnp.maximum(acc + b_ref[...], 0.0)
            if li == 0:
                yv = (y * m_ref[...]).astype(jnp.bfloat16)
                s1_ref[pl.ds((i * tr + 2) * Wp, m2), :] = yv
            else:
                pv = _pool2x2(y, tr=tr, Wp=Wp, tco=128, pad_out=True)
                o_ref[0, pl.ds((i * 14 + 2) * 58, 14 * 58), :] = (
                    pv.reshape(14 * 58, 128).astype(o_ref.dtype))


def _chain112(x, w2, b2, w3, b3):
    N = x.shape[0]
    wm2 = w2.reshape(1, 9 * 64, 128).astype(jnp.bfloat16)
    wm3 = w3.reshape(1, 9 * 128, 128).astype(jnp.bfloat16)
    grid_spec = pltpu.PrefetchScalarGridSpec(
        num_scalar_prefetch=0,
        grid=(N,),
        in_specs=[
            pl.BlockSpec((1, 116 * 114, 64), lambda n: (n, 0, 0)),
            pl.BlockSpec((1, 9 * 64, 128), lambda n: (0, 0, 0)),
            pl.BlockSpec((1, 128), lambda n: (0, 0)),
            pl.BlockSpec((1, 9 * 128, 128), lambda n: (0, 0, 0)),
            pl.BlockSpec((1, 128), lambda n: (0, 0)),
            pl.BlockSpec((28 * 114, 1), lambda n: (0, 0)),
        ],
        out_specs=pl.BlockSpec((1, 60 * 58, 128), lambda n: (n, 0, 0)),
        scratch_shapes=[pltpu.VMEM((116 * 114, 128), jnp.bfloat16)],
    )
    return pl.pallas_call(
        _chain112_body,
        out_shape=jax.ShapeDtypeStruct((N, 60 * 58, 128), jnp.bfloat16),
        grid_spec=grid_spec,
        compiler_params=pltpu.CompilerParams(
            dimension_semantics=("parallel",)),
    )(x, wm2, b2.reshape(1, 128).astype(jnp.float32),
      wm3, b3.reshape(1, 128).astype(jnp.float32), _colmask(28 * 114, 114))


def _chain56_body(x_ref, w0_ref, b0_ref, w1_ref, b1_ref, w2_ref, b2_ref,
                  m_ref, o_ref, s1_ref, s2_ref):
    Wp = 58
    tr = 28
    m2 = tr * Wp
    H = 56
    for li in range(3):
        if li == 0:
            src, w_ref, b_ref, dst = x_ref, w0_ref, b0_ref, s1_ref
        elif li == 1:
            src, w_ref, b_ref, dst = s1_ref, w1_ref, b1_ref, s2_ref
        else:
            src, w_ref, b_ref, dst = s2_ref, w2_ref, b2_ref, None
        if dst is not None:
            zrow = jnp.zeros((2 * Wp, 256), jnp.bfloat16)
            dst[pl.ds(0, 2 * Wp), :] = zrow
            dst[pl.ds((H + 2) * Wp, 2 * Wp), :] = zrow
        else:
            Wp2 = 30
            zr2 = jnp.zeros((2 * Wp2, 256), o_ref.dtype)
            o_ref[0, pl.ds(0, 2 * Wp2), :] = zr2
            o_ref[0, pl.ds(30 * Wp2, 2 * Wp2), :] = zr2
        for i in range(2):
            base = i * m2

            def _ld(off):
                if li == 0:
                    return x_ref[0, pl.ds(base + off, m2), :]
                return src[pl.ds(base + off, m2), :]

            if li == 0:
                lhs = jnp.concatenate(
                    [_ld((dy + 1) * Wp + dx - 1)
                     for dy in range(3) for dx in range(3)], axis=-1)
                acc = jnp.dot(lhs, w_ref[0],
                              preferred_element_type=jnp.float32)
            else:
                acc = jnp.zeros((m2, 256), jnp.float32)
                for dy in range(3):
                    for dx in range(3):
                        acc = acc + jnp.dot(
                            _ld((dy + 1) * Wp + dx - 1),
                            w_ref[dy * 3 + dx],
                            preferred_element_type=jnp.float32)
            y = jnp.maximum(acc + b_ref[...], 0.0)
            if dst is not None:
                yv = (y * m_ref[...]).astype(jnp.bfloat16)
                dst[pl.ds((i * tr + 2) * Wp, m2), :] = yv
            else:
                pv = _pool2x2(y, tr=tr, Wp=Wp, tco=256, pad_out=True)
                o_ref[0, pl.ds((i * 14 + 2) * 30, 14 * 30), :] = (
                    pv.reshape(14 * 30, 256).astype(o_ref.dtype))


def _chain56(x, w4, b4, w5, b5, w6, b6):
    N = x.shape[0]
    wm4 = w4.reshape(1, 9 * 128, 256).astype(jnp.bfloat16)
    args = [wm4, b4.reshape(1, 256).astype(jnp.float32)]
    specs = [pl.BlockSpec((1, 9 * 128, 256), lambda n: (0, 0, 0)),
             pl.BlockSpec((1, 256), lambda n: (0, 0))]
    for w, b in ((w5, b5), (w6, b6)):
        args.append(w.reshape(9, 256, 256).astype(jnp.bfloat16))
        args.append(b.reshape(1, 256).astype(jnp.float32))
        specs.append(pl.BlockSpec((9, 256, 256), lambda n: (0, 0, 0)))
        specs.append(pl.BlockSpec((1, 256), lambda n: (0, 0)))
    grid_spec = pltpu.PrefetchScalarGridSpec(
        num_scalar_prefetch=0,
        grid=(N,),
        in_specs=[pl.BlockSpec((1, 60 * 58, 128), lambda n: (n, 0, 0))]
        + specs + [pl.BlockSpec((28 * 58, 1), lambda n: (0, 0))],
        out_specs=pl.BlockSpec((1, 32 * 30, 256), lambda n: (n, 0, 0)),
        scratch_shapes=[pltpu.VMEM((60 * 58, 256), jnp.bfloat16),
                        pltpu.VMEM((60 * 58, 256), jnp.bfloat16)],
    )
    return pl.pallas_call(
        _chain56_body,
        out_shape=jax.ShapeDtypeStruct((N, 32 * 30, 256), jnp.bfloat16),
        grid_spec=grid_spec,
        compiler_params=pltpu.CompilerParams(
            dimension_semantics=("parallel",)),
    )(x, *args, _colmask(28 * 58, 58))


def _chain28_body(x_ref, w0_ref, b0_ref, w1_ref, b1_ref, w2_ref, b2_ref,
                  m_ref, o_ref):
    Wp = 30
    m2 = 28 * Wp
    x = x_ref[0]
    for li, (w_ref, b_ref) in enumerate(((w0_ref, b0_ref), (w1_ref, b1_ref),
                                         (w2_ref, b2_ref))):
        acc = jnp.zeros((m2, 512), jnp.float32)
        for dy in range(3):
            for dx in range(3):
                off = (dy + 1) * Wp + dx - 1
                acc = acc + jnp.dot(
                    x[off:off + m2, :],
                    w_ref[dy * 3 + dx], preferred_element_type=jnp.float32)
        y = jnp.maximum(acc + b_ref[...], 0.0)
        if li < 2:
            yv = (y * m_ref[...]).astype(jnp.bfloat16)
            x = jnp.pad(yv, ((2 * Wp, 2 * Wp), (0, 0)))
        else:
            Wp2 = 16
            zrow = jnp.zeros((2 * Wp2, 512), o_ref.dtype)
            o_ref[0, pl.ds(0, 2 * Wp2), :] = zrow
            o_ref[0, pl.ds(16 * Wp2, 2 * Wp2), :] = zrow
            pv = _pool2x2(y, tr=28, Wp=Wp, tco=512, pad_out=True)
            o_ref[0, pl.ds(2 * Wp2, 14 * Wp2), :] = (
                pv.reshape(14 * Wp2, 512).astype(o_ref.dtype))


def _chain28(x, w7, b7, w8, b8, w9, b9):
    N = x.shape[0]
    args = []
    specs = []
    for w, b in ((w7, b7), (w8, b8), (w9, b9)):
        cin = w.shape[2]
        args.append(w.reshape(9, cin, 512).astype(jnp.bfloat16))
        args.append(b.reshape(1, 512).astype(jnp.float32))
        specs.append(pl.BlockSpec((9, cin, 512), lambda n: (0, 0, 0)))
        specs.append(pl.BlockSpec((1, 512), lambda n: (0, 0)))
    return pl.pallas_call(
        _chain28_body,
        out_shape=jax.ShapeDtypeStruct((N, 18 * 16, 512), jnp.bfloat16),
        grid=(N,),
        in_specs=[pl.BlockSpec((1, 32 * 30, 256), lambda n: (n, 0, 0))]
        + specs + [pl.BlockSpec((28 * 30, 1), lambda n: (0, 0))],
        out_specs=pl.BlockSpec((1, 18 * 16, 512), lambda n: (n, 0, 0)),
        compiler_params=pltpu.CompilerParams(
            dimension_semantics=("parallel",)),
    )(x, *args, _colmask(28 * 30, 30))


def _chain14(x, w10, b10, w11, b11, w12, b12):
    N = x.shape[0]
    args = []
    for w, b in ((w10, b10), (w11, b11), (w12, b12)):
        args.append(w.reshape(9, 512, 512).astype(jnp.bfloat16))
        args.append(b.reshape(1, 512).astype(jnp.float32))
    wspec = pl.BlockSpec((9, 512, 512), lambda n: (0, 0, 0))
    bspec = pl.BlockSpec((1, 512), lambda n: (0, 0))
    return pl.pallas_call(
        _chain14_body,
        out_shape=jax.ShapeDtypeStruct((N, 49, 512), jnp.bfloat16),
        grid=(N,),
        in_specs=[
            pl.BlockSpec((1, 18 * 16, 512), lambda n: (n, 0, 0)),
            wspec, bspec, wspec, bspec, wspec, bspec,
            pl.BlockSpec((14 * 16, 1), lambda n: (0, 0)),
        ],
        out_specs=pl.BlockSpec((1, 49, 512), lambda n: (n, 0, 0)),
        compiler_params=pltpu.CompilerParams(
            dimension_semantics=("parallel",)),
    )(x, *args, _colmask(14 * 16, 16))


def _fc_body(x_ref, w_ref, b_ref, s_ref, o_ref, *, relu):
    y = jnp.dot(x_ref[...], w_ref[...],
                preferred_element_type=jnp.float32) + b_ref[...]
    if relu:
        y = jnp.maximum(y, 0.0)
    o_ref[...] = (y * s_ref[...]).astype(o_ref.dtype)


def _fc(x, w, b, scale, *, relu, tn, out_dtype):
    B, K = x.shape
    Nout = w.shape[1]
    return pl.pallas_call(
        functools.partial(_fc_body, relu=relu),
        out_shape=jax.ShapeDtypeStruct((B, Nout), out_dtype),
        grid=(Nout // tn,),
        in_specs=[
            pl.BlockSpec((B, K), lambda j: (0, 0)),
            pl.BlockSpec((K, tn), lambda j: (0, j)),
            pl.BlockSpec((1, tn), lambda j: (0, j)),
            pl.BlockSpec((B, tn), lambda j: (0, j)),
        ],
        out_specs=pl.BlockSpec((B, tn), lambda j: (0, j)),
        compiler_params=pltpu.CompilerParams(
            dimension_semantics=("parallel",)),
    )(x, w, b.reshape(1, Nout).astype(jnp.float32), scale.astype(jnp.float32))


def kernel(conv0_w, conv0_b, conv1_w, conv1_b, conv2_w, conv2_b, conv3_w,
           conv3_b, conv4_w, conv4_b, conv5_w, conv5_b, conv6_w, conv6_b,
           conv7_w, conv7_b, conv8_w, conv8_b, conv9_w, conv9_b, conv10_w,
           conv10_b, conv11_w, conv11_b, conv12_w, conv12_b, fc1_w, fc1_b,
           fc2_w, fc2_b, fc3_w, fc3_b, x_nhwc, drop_key):
    convs = [(conv0_w, conv0_b), (conv1_w, conv1_b), (conv2_w, conv2_b),
             (conv3_w, conv3_b), (conv4_w, conv4_b), (conv5_w, conv5_b),
             (conv6_w, conv6_b), (conv7_w, conv7_b), (conv8_w, conv8_b),
             (conv9_w, conv9_b), (conv10_w, conv10_b), (conv11_w, conv11_b),
             (conv12_w, conv12_b)]
    N, H, W, _ = x_nhwc.shape

    x = _conv0(x_nhwc, conv0_w, conv0_b)
    x = jnp.pad(x.reshape(N, 224, 226, 64), ((0, 0), (2, 2), (0, 0), (0, 0)))
    x = _conv_banded(_bands(x, 56), conv1_w, conv1_b, H=224, W=224, tr=56,
                     pool=True, group=9)
    x = jnp.pad(x.reshape(N, 112, 114, 64), ((0, 0), (2, 2), (0, 0), (0, 0)))
    x = x.reshape(N, 116 * 114, 64)

    x = _chain112(x, conv2_w, conv2_b, conv3_w, conv3_b)
    x = _chain56(x, conv4_w, conv4_b, conv5_w, conv5_b, conv6_w, conv6_b)
    x = _chain28(x, conv7_w, conv7_b, conv8_w, conv8_b, conv9_w, conv9_b)
    x = _chain14(x, conv10_w, conv10_b, conv11_w, conv11_b,
                 conv12_w, conv12_b)

    x = jnp.transpose(x, (0, 2, 1)).reshape(N, 512 * 7 * 7)

    k1, k2 = jax.random.split(drop_key)
    s1 = jax.random.bernoulli(k1, 0.5, (N, 4096)).astype(jnp.float32) / 0.5
    s2 = jax.random.bernoulli(k2, 0.5, (N, 4096)).astype(jnp.float32) / 0.5

    x = _fc(x, fc1_w, fc1_b, s1, relu=True, tn=256, out_dtype=jnp.bfloat16)
    x = _fc(x, fc2_w, fc2_b, s2, relu=True, tn=256, out_dtype=jnp.bfloat16)
    ones = jnp.ones((N, fc3_w.shape[1]), jnp.float32)
    x = _fc(x, fc3_w, fc3_b, ones, relu=False, tn=fc3_w.shape[1],
            out_dtype=jnp.float32)
    return x
